# baseline (speedup 1.0000x reference)
import functools

import jax
import jax.numpy as jnp
from jax import lax
from jax.experimental import pallas as pl
from jax.experimental.pallas import tpu as pltpu
from jax.experimental.pallas import tpu_sc as plsc

_NC = 2
_NS = 16
_NW = _NC * _NS
_FW = 72


def _sc_hist_body(rows, crows, ls, lg, sports_hbm, gym_hbm, gender_hbm,
                  pref_hbm, age_hbm, f_hbm, s_v, g_v, gd_v, pf_v, age_v, f_v,
                  sem):
  wid = lax.axis_index("s") * _NC + lax.axis_index("c")
  zeros = jnp.zeros((16,), jnp.float32)
  ones = jnp.ones((16,), jnp.float32)
  iota = lax.iota(jnp.int32, 16)

  for c in range(rows // crows):
    base = wid * rows + c * crows

    cp_s = pltpu.async_copy(sports_hbm.at[pl.ds(base, crows), :], s_v, sem)
    cp_g = pltpu.async_copy(gym_hbm.at[pl.ds(base, crows), :], g_v, sem)
    cp_gd = pltpu.async_copy(gender_hbm.at[pl.ds(base, crows)], gd_v, sem)
    cp_pf = pltpu.async_copy(pref_hbm.at[pl.ds(base, crows)], pf_v, sem)
    cp_age = pltpu.async_copy(age_hbm.at[pl.ds(base, crows)], age_v, sem)

    def zero_body(i, carry):
      for u in range(16):
        f_v[pl.ds(i * 256 + u * 16, 16)] = zeros
      return carry

    lax.fori_loop(0, crows * _FW // 256, zero_body, 0)

    cp_s.wait()
    cp_g.wait()
    cp_gd.wait()
    cp_pf.wait()
    cp_age.wait()

    def group_body(g, carry):
      row_vec = iota + g * 16
      lane_base = row_vec * _FW
      lane_base52 = lane_base + 52
      for l in range(ls):
        ids = plsc.load_gather(s_v, [row_vec, jnp.full((16,), l, jnp.int32)])
        plsc.addupdate_scatter(f_v, [ids + lane_base], ones)
      for l in range(lg):
        ids = plsc.load_gather(g_v, [row_vec, jnp.full((16,), l, jnp.int32)])
        plsc.addupdate_scatter(f_v, [ids + lane_base52], ones)
      gd = gd_v[pl.ds(g * 16, 16)]
      plsc.addupdate_scatter(f_v, [gd + (lane_base + 60)], ones)
      pf = pf_v[pl.ds(g * 16, 16)]
      plsc.addupdate_scatter(f_v, [pf + (lane_base + 62)], ones)
      age = age_v[pl.ds(g * 16, 16)]
      plsc.addupdate_scatter(f_v, [lane_base + 64], age)
      return carry

    lax.fori_loop(0, crows // 16, group_body, 0)

    pltpu.sync_copy(f_v, f_hbm.at[pl.ds(base * _FW, crows * _FW)])


def _sc_hist(sports_ids, gym_days, gender, pref, age, B):
  rows = B // _NW
  crows = rows // 2
  ls = sports_ids.shape[1]
  lg = gym_days.shape[1]
  mesh = plsc.VectorSubcoreMesh(core_axis_name="c", subcore_axis_name="s",
                                num_cores=_NC, num_subcores=_NS)
  return pl.kernel(
      functools.partial(_sc_hist_body, rows, crows, ls, lg),
      out_type=jax.ShapeDtypeStruct((B * _FW,), jnp.float32),
      mesh=mesh,
      scratch_types=[
          pltpu.VMEM((crows, ls), jnp.int32),
          pltpu.VMEM((crows, lg), jnp.int32),
          pltpu.VMEM((crows,), jnp.int32),
          pltpu.VMEM((crows,), jnp.int32),
          pltpu.VMEM((crows,), jnp.float32),
          pltpu.VMEM((crows * _FW,), jnp.float32),
          pltpu.SemaphoreType.DMA,
      ],
      compiler_params=pltpu.CompilerParams(needs_layout_passes=False),
  )(sports_ids, gym_days, gender, pref, age)


def _tc_mlp_body(f_ref, st_ref, gt_ref, gyt_ref, w1a_ref, w1gd_ref, w1pf_ref,
                 w1gy_ref, wage_ref, b1_ref, w2_ref, b2_ref, out_ref):
  f32 = jnp.float32
  t_sport = jnp.dot(st_ref[...], w1a_ref[...], preferred_element_type=f32)
  t_gym = jnp.dot(gyt_ref[...], w1gy_ref[...], preferred_element_type=f32)
  a_gd = jnp.dot(gt_ref[...], w1gd_ref[...], preferred_element_type=f32)
  a_pf = jnp.dot(gt_ref[...], w1pf_ref[...], preferred_element_type=f32)
  wage = wage_ref[...]
  m = jnp.concatenate(
      [t_sport * (1.0 / 20.0), t_gym * (1.0 / 7.0), a_gd, a_pf,
       wage * (1.0 / 6.5), jnp.zeros((7, 64), f32)], axis=0)
  b1 = b1_ref[...] - (19.0 / 6.5) * wage
  h = jnp.dot(f_ref[...], m, preferred_element_type=f32) + b1
  h = jnp.maximum(h, 0.0)
  out_ref[...] = jnp.dot(h, w2_ref[...], preferred_element_type=f32) + b2_ref[...]


def _tc_mlp(F, sport_table, gender_table, gym_table,
            w1a, w1gd, w1pf, w1gy, w_age, b1, W2, b2):
  B = F.shape[0]
  blk = 4096
  grid = (B // blk,)
  rep_spec = lambda shape: pl.BlockSpec(shape, lambda i: (0,) * len(shape))
  in_specs = [
      pl.BlockSpec((blk, _FW), lambda i: (i, 0)),
      rep_spec(sport_table.shape),
      rep_spec(gender_table.shape),
      rep_spec(gym_table.shape),
      rep_spec(w1a.shape),
      rep_spec(w1gd.shape),
      rep_spec(w1pf.shape),
      rep_spec(w1gy.shape),
      rep_spec(w_age.shape),
      rep_spec(b1.shape),
      rep_spec(W2.shape),
      rep_spec(b2.shape),
  ]
  return pl.pallas_call(
      _tc_mlp_body,
      grid=grid,
      in_specs=in_specs,
      out_specs=pl.BlockSpec((blk, 32), lambda i: (i, 0)),
      out_shape=jax.ShapeDtypeStruct((B, 32), jnp.float32),
      compiler_params=pltpu.CompilerParams(
          dimension_semantics=("parallel",)),
  )(F, sport_table, gender_table, gym_table,
    w1a, w1gd, w1pf, w1gy, w_age, b1, W2, b2)


def kernel(sports_ids, age, gender, preferred_gender, gym_days,
           sport_table, gender_table, gym_table, W1, b1, W2, b2):
  B = sports_ids.shape[0]
  f_flat = _sc_hist(
      sports_ids.astype(jnp.int32), gym_days.astype(jnp.int32),
      gender.astype(jnp.int32).reshape(B), preferred_gender.astype(jnp.int32).reshape(B),
      age.reshape(B), B)
  F = f_flat.reshape(B, _FW)
  out = _tc_mlp(
      F, sport_table, gender_table, gym_table,
      W1[0:10], W1[10:14], W1[14:18], W1[18:22], W1[22:23],
      b1.reshape(1, 64), W2, b2.reshape(1, 32))
  return out


# speedup vs baseline: 2.1986x; 2.1986x over previous
import functools

import jax
import jax.numpy as jnp
from jax import lax
from jax.experimental import pallas as pl
from jax.experimental.pallas import tpu as pltpu
from jax.experimental.pallas import tpu_sc as plsc

_NC = 2
_NS = 16
_NW = _NC * _NS
_FW = 128
_FS = 65


def _sc_hist_body(rows, crows, ls, lg, sports_hbm, gym_hbm, gender_hbm,
                  pref_hbm, age_hbm, f_hbm, s_v, g_v, gd_v, pf_v, age_v,
                  f_v0, f_v1, sem_in, sem_out):
  wid = lax.axis_index("s") * _NC + lax.axis_index("c")
  zeros = jnp.zeros((16,), jnp.float32)
  ones = jnp.ones((16,), jnp.float32)
  iota = lax.iota(jnp.int32, 16)
  f_bufs = [f_v0, f_v1]
  out_handles = []
  n_chunks = rows // crows

  for c in range(n_chunks):
    base = wid * rows + c * crows
    f_v = f_bufs[c % 2]
    if c >= 2:
      out_handles[c - 2].wait()

    cp_s = pltpu.async_copy(
        sports_hbm.at[:, pl.ds(base, crows)], s_v, sem_in)
    cp_g = pltpu.async_copy(gym_hbm.at[:, pl.ds(base, crows)], g_v, sem_in)
    cp_gd = pltpu.async_copy(gender_hbm.at[pl.ds(base, crows)], gd_v, sem_in)
    cp_pf = pltpu.async_copy(pref_hbm.at[pl.ds(base, crows)], pf_v, sem_in)
    cp_age = pltpu.async_copy(age_hbm.at[pl.ds(base, crows)], age_v, sem_in)

    def zero_body(r, carry):
      for u in range(4):
        f_v[pl.ds(r * _FW + u * 16, 16)] = zeros
      return carry

    lax.fori_loop(0, crows, zero_body, 0, unroll=4)

    cp_s.wait()
    cp_g.wait()
    cp_gd.wait()
    cp_pf.wait()
    cp_age.wait()

    def group_body(g, carry):
      row_vec = iota + g * 16
      rb = row_vec * _FW
      rb52 = rb + 52
      work = [(s_v, l, rb) for l in range(ls)] + \
             [(g_v, l, rb52) for l in range(lg)]
      for blk in range(0, len(work), 8):
        chunk = work[blk:blk + 8]
        vals = [ref[l, pl.ds(g * 16, 16)] for ref, l, _ in chunk]
        idxs = [v + b for v, (_, _, b) in zip(vals, chunk)]
        for ix in idxs:
          plsc.addupdate_scatter(f_v, [ix], ones)
      gd = gd_v[pl.ds(g * 16, 16)]
      pf = pf_v[pl.ds(g * 16, 16)]
      age = age_v[pl.ds(g * 16, 16)]
      plsc.addupdate_scatter(f_v, [gd + (rb + 60)], ones)
      plsc.addupdate_scatter(f_v, [pf + (rb + 62)], ones)
      plsc.store_scatter(f_v, [rb + 64], age)
      return carry

    lax.fori_loop(0, crows // 16, group_body, 0)

    out_handles.append(pltpu.async_copy(
        f_v, f_hbm.at[pl.ds(base * _FW, crows * _FW)], sem_out))

  for h in out_handles[max(0, n_chunks - 2):]:
    h.wait()


def _sc_hist(sports_ids, gym_days, gender, pref, age, B):
  rows = B // _NW
  crows = rows // 4
  ls = sports_ids.shape[0]
  lg = gym_days.shape[0]
  mesh = plsc.VectorSubcoreMesh(core_axis_name="c", subcore_axis_name="s",
                                num_cores=_NC, num_subcores=_NS)
  return pl.kernel(
      functools.partial(_sc_hist_body, rows, crows, ls, lg),
      out_type=jax.ShapeDtypeStruct((B * _FW,), jnp.float32),
      mesh=mesh,
      scratch_types=[
          pltpu.VMEM((ls, crows), jnp.int32),
          pltpu.VMEM((lg, crows), jnp.int32),
          pltpu.VMEM((crows,), jnp.int32),
          pltpu.VMEM((crows,), jnp.int32),
          pltpu.VMEM((crows,), jnp.float32),
          pltpu.VMEM((crows * _FW,), jnp.float32),
          pltpu.VMEM((crows * _FW,), jnp.float32),
          pltpu.SemaphoreType.DMA,
          pltpu.SemaphoreType.DMA,
      ],
      compiler_params=pltpu.CompilerParams(needs_layout_passes=False),
  )(sports_ids, gym_days, gender, pref, age)


def _tc_mlp_body(f_ref, st_ref, gt_ref, gyt_ref, w1a_ref, w1gd_ref, w1pf_ref,
                 w1gy_ref, wage_ref, b1_ref, w2_ref, b2_ref, out_ref):
  f32 = jnp.float32
  f = f_ref[...]
  lane = lax.broadcasted_iota(jnp.int32, (1, _FW), 1)
  f = jnp.where(lane < _FS, f, 0.0)
  t_sport = jnp.dot(st_ref[...], w1a_ref[...], preferred_element_type=f32)
  t_gym = jnp.dot(gyt_ref[...], w1gy_ref[...], preferred_element_type=f32)
  a_gd = jnp.dot(gt_ref[...], w1gd_ref[...], preferred_element_type=f32)
  a_pf = jnp.dot(gt_ref[...], w1pf_ref[...], preferred_element_type=f32)
  wage = wage_ref[...]
  m = jnp.concatenate(
      [t_sport * (1.0 / 20.0), t_gym * (1.0 / 7.0), a_gd, a_pf,
       wage * (1.0 / 6.5), jnp.zeros((_FW - _FS, 64), f32)], axis=0)
  b1 = b1_ref[...] - (19.0 / 6.5) * wage
  h = jnp.maximum(jnp.dot(f, m, preferred_element_type=f32) + b1, 0.0)
  out_t = lax.dot_general(w2_ref[...], h, (((0,), (1,)), ((), ())),
                          preferred_element_type=f32)
  out_ref[...] = out_t + b2_ref[...]


def _tc_mlp(F, sport_table, gender_table, gym_table,
            w1a, w1gd, w1pf, w1gy, w_age, b1, W2, b2):
  B = F.shape[0]
  blk = 8192
  grid = (B // blk,)
  rep_spec = lambda shape: pl.BlockSpec(shape, lambda i: (0,) * len(shape))
  in_specs = [
      pl.BlockSpec((blk, _FW), lambda i: (i, 0)),
      rep_spec(sport_table.shape),
      rep_spec(gender_table.shape),
      rep_spec(gym_table.shape),
      rep_spec(w1a.shape),
      rep_spec(w1gd.shape),
      rep_spec(w1pf.shape),
      rep_spec(w1gy.shape),
      rep_spec(w_age.shape),
      rep_spec(b1.shape),
      rep_spec(W2.shape),
      rep_spec(b2.shape),
  ]
  return pl.pallas_call(
      _tc_mlp_body,
      grid=grid,
      in_specs=in_specs,
      out_specs=pl.BlockSpec((32, blk), lambda i: (0, i)),
      out_shape=jax.ShapeDtypeStruct((32, B), jnp.float32),
      compiler_params=pltpu.CompilerParams(
          dimension_semantics=("parallel",)),
  )(F, sport_table, gender_table, gym_table,
    w1a, w1gd, w1pf, w1gy, w_age, b1, W2, b2)


def kernel(sports_ids, age, gender, preferred_gender, gym_days,
           sport_table, gender_table, gym_table, W1, b1, W2, b2):
  B = sports_ids.shape[0]
  F = _sc_hist(
      sports_ids.astype(jnp.int32).T, gym_days.astype(jnp.int32).T,
      gender.astype(jnp.int32).reshape(B),
      preferred_gender.astype(jnp.int32).reshape(B),
      age.reshape(B), B).reshape(B, _FW)
  out_t = _tc_mlp(
      F, sport_table, gender_table, gym_table,
      W1[0:10], W1[10:14], W1[14:18], W1[18:22], W1[22:23],
      b1.reshape(1, 64), W2, b2.reshape(32, 1))
  return out_t.T
